# baseline (device time: 14781 ns/iter reference)
import jax
import jax.numpy as jnp
from jax import lax
from jax.experimental import pallas as pl
from jax.experimental.pallas import tpu as pltpu

N_DEV = 8
E_LOCAL = 2


def kernel(x, router_W, route_idx, expert_W, shared_W):
    T, D = x.shape
    _, _, H = expert_W.shape
    E = N_DEV * E_LOCAL

    def body(x_ref, rw_ref, idx_ref, ew_ref, sw_ref, out_ref,
             sendbuf_ref, comm_ref, send_sems, recv_sems):
        my = lax.axis_index("i")

        barrier_sem = pltpu.get_barrier_semaphore()
        for k in range(1, N_DEV):
            pl.semaphore_signal(
                barrier_sem, inc=1,
                device_id=(lax.rem(my + k, N_DEV),),
                device_id_type=pl.DeviceIdType.MESH,
            )
        pl.semaphore_wait(barrier_sem, N_DEV - 1)

        ew_bf = ew_ref[...].astype(jnp.bfloat16).reshape(E_LOCAL * D, H)
        sendbuf_ref[...] = ew_bf
        for k in range(1, N_DEV):
            rdma = pltpu.make_async_remote_copy(
                src_ref=sendbuf_ref,
                dst_ref=comm_ref.at[k - 1],
                send_sem=send_sems.at[k - 1],
                recv_sem=recv_sems.at[k - 1],
                device_id=(lax.rem(my + k, N_DEV),),
                device_id_type=pl.DeviceIdType.MESH,
            )
            rdma.start()

        x_bf = x_ref[...].astype(jnp.bfloat16)
        scores = jnp.dot(x_ref[...], rw_ref[...],
                         preferred_element_type=jnp.float32)
        s_max = jnp.max(scores, axis=-1, keepdims=True)
        p = jnp.exp(scores - s_max)
        probs = p / jnp.sum(p, axis=-1, keepdims=True)
        eidx = lax.broadcasted_iota(jnp.int32, (T, E), 1)
        coef = jnp.where(idx_ref[...] == eidx, probs, 0.0)
        coef_rot = pltpu.roll(
            coef.astype(jnp.bfloat16),
            jnp.mod(E - E_LOCAL * my, E), 1)

        acc = jnp.dot(x_bf, sw_ref[...].astype(jnp.bfloat16),
                      preferred_element_type=jnp.float32)

        def expert_pair(rel, w):
            c = coef_rot[:, rel:rel + E_LOCAL]
            xcat = jnp.concatenate(
                [x_bf * c[:, j:j + 1] for j in range(E_LOCAL)], axis=1)
            return jnp.dot(xcat, w, preferred_element_type=jnp.float32)

        acc = acc + expert_pair(0, sendbuf_ref[...])

        for k in range(1, N_DEV):
            recv = pltpu.make_async_remote_copy(
                src_ref=sendbuf_ref,
                dst_ref=comm_ref.at[k - 1],
                send_sem=send_sems.at[k - 1],
                recv_sem=recv_sems.at[k - 1],
                device_id=(0,),
                device_id_type=pl.DeviceIdType.MESH,
            )
            recv.wait_recv()
            acc = acc + expert_pair(E - E_LOCAL * k, comm_ref[k - 1])

        out_ref[...] = acc

        for k in range(1, N_DEV):
            send = pltpu.make_async_remote_copy(
                src_ref=sendbuf_ref,
                dst_ref=comm_ref.at[k - 1],
                send_sem=send_sems.at[k - 1],
                recv_sem=recv_sems.at[k - 1],
                device_id=(0,),
                device_id_type=pl.DeviceIdType.MESH,
            )
            send.wait_send()

    return pl.pallas_call(
        body,
        out_shape=jax.ShapeDtypeStruct((T, H), jnp.float32),
        in_specs=[pl.BlockSpec(memory_space=pltpu.VMEM)] * 5,
        out_specs=pl.BlockSpec(memory_space=pltpu.VMEM),
        scratch_shapes=[
            pltpu.VMEM((E_LOCAL * D, H), jnp.bfloat16),
            pltpu.VMEM((N_DEV - 1, E_LOCAL * D, H), jnp.bfloat16),
            pltpu.SemaphoreType.DMA((N_DEV - 1,)),
            pltpu.SemaphoreType.DMA((N_DEV - 1,)),
        ],
        compiler_params=pltpu.CompilerParams(collective_id=0),
    )(x, router_W, route_idx, expert_W, shared_W)
